# Initial kernel scaffold; baseline (speedup 1.0000x reference)
import jax
import jax.numpy as jnp
from jax.experimental import pallas as pl


def kernel(context, forecast, forecast_mask):
    raise NotImplementedError("write your pallas kernel here")


# speedup vs baseline: 1.5705x; 1.5705x over previous
import jax
import jax.numpy as jnp
from jax.experimental import pallas as pl

_B = 16384
_H = 512
_BLK = 512


def _body(mask_ref, u_ref, pos_ref, probs_ref):
    m = mask_ref[...].astype(jnp.int32)
    valid = jnp.sum(m, axis=1, keepdims=True)
    validf = valid.astype(jnp.float32)
    u = u_ref[...]
    idx = jnp.floor(u * validf).astype(jnp.int32)
    idx = jnp.minimum(idx, jnp.maximum(valid - 1, 0))
    cum = m
    sh = 1
    while sh < m.shape[1]:
        z = jnp.zeros((cum.shape[0], sh), cum.dtype)
        cum = cum + jnp.concatenate([z, cum[:, :-sh]], axis=1)
        sh *= 2
    pos = jnp.sum((cum <= idx).astype(jnp.int32), axis=1, keepdims=True)
    pos = jnp.where(valid > 0, pos, 0)
    pos_ref[...] = pos
    probs_ref[...] = jnp.zeros_like(probs_ref)


def kernel(context, forecast, forecast_mask):
    del context, forecast
    B, H = forecast_mask.shape
    u = jax.random.uniform(jax.random.key(42), (B,)).reshape(B, 1)
    grid = (B // _BLK,)
    pos, probs = pl.pallas_call(
        _body,
        grid=grid,
        in_specs=[
            pl.BlockSpec((_BLK, H), lambda i: (i, 0)),
            pl.BlockSpec((_BLK, 1), lambda i: (i, 0)),
        ],
        out_specs=[
            pl.BlockSpec((_BLK, 1), lambda i: (i, 0)),
            pl.BlockSpec((_BLK, H), lambda i: (i, 0)),
        ],
        out_shape=[
            jax.ShapeDtypeStruct((B, 1), jnp.int32),
            jax.ShapeDtypeStruct((B, H), jnp.float32),
        ],
    )(forecast_mask, u)
    return pos.reshape(B), probs
